# baseline (speedup 1.0000x reference)
import functools

import jax
import jax.numpy as jnp
from jax import lax
from jax.experimental import pallas as pl
from jax.experimental.pallas import tpu as pltpu
from jax.experimental.pallas import tpu_sc as plsc

_NC = 2
_NS = 16
_NW = _NC * _NS


def _sc_gather_fn(n_tokens, hidden, chunk):
    n_per_w = n_tokens // _NW
    n_chunks = n_per_w // chunk

    mesh = plsc.VectorSubcoreMesh(core_axis_name="c", subcore_axis_name="s")

    @functools.partial(
        pl.kernel,
        out_type=jax.ShapeDtypeStruct((n_tokens, hidden), jnp.float32),
        mesh=mesh,
        scratch_types=[
            pltpu.VMEM((chunk,), jnp.int32),
            pltpu.VMEM((chunk, hidden), jnp.float32),
            pltpu.SemaphoreType.DMA,
        ],
        compiler_params=pltpu.CompilerParams(use_tc_tiling_on_sc=True),
    )
    def sc_gather(idx_hbm, table_hbm, x_hbm, idx_v, rows_v, sem):
        wid = lax.axis_index("s") * _NC + lax.axis_index("c")
        base = wid * n_per_w
        for c in range(n_chunks):
            off = base + c * chunk
            pltpu.sync_copy(idx_hbm.at[pl.ds(off, chunk)], idx_v)
            pltpu.async_copy(table_hbm.at[idx_v], rows_v, sem).wait()
            pltpu.sync_copy(rows_v, x_hbm.at[pl.ds(off, chunk)])

    return sc_gather


def _tc_matmul_body(x_ref, w_ref, b_ref, o_ref):
    o_ref[...] = (
        jnp.dot(x_ref[...], w_ref[...], preferred_element_type=jnp.float32)
        + b_ref[...]
    )


def _tc_matmul(x, w, b2d, block_m):
    n_tokens, hidden = x.shape
    vocab = w.shape[1]
    grid = (n_tokens // block_m,)
    return pl.pallas_call(
        _tc_matmul_body,
        grid=grid,
        in_specs=[
            pl.BlockSpec((block_m, hidden), lambda i: (i, 0)),
            pl.BlockSpec((hidden, vocab), lambda i: (0, 0)),
            pl.BlockSpec((1, vocab), lambda i: (0, 0)),
        ],
        out_specs=pl.BlockSpec((block_m, vocab), lambda i: (i, 0)),
        out_shape=jax.ShapeDtypeStruct((n_tokens, vocab), jnp.float32),
    )(x, w, b2d)


@jax.jit
def kernel(input_ids, embedding, W, b):
    bsz, seqlen = input_ids.shape
    vocab, hidden = embedding.shape
    n_tokens = bsz * seqlen

    ids = input_ids.reshape(-1).astype(jnp.int32)
    x = _sc_gather_fn(n_tokens, hidden, chunk=80)(ids, embedding)
    logits = _tc_matmul(x, W, b.reshape(1, -1), block_m=512)
    return logits.reshape(bsz, seqlen, vocab)


# speedup vs baseline: 3.2509x; 3.2509x over previous
import functools

import jax
import jax.numpy as jnp
from jax import lax
from jax.experimental import pallas as pl
from jax.experimental.pallas import tpu as pltpu
from jax.experimental.pallas import tpu_sc as plsc

_NC = 2
_NS = 16
_NW = _NC * _NS


def _sc_gather_fn(n_tokens, hidden, chunk):
    n_per_w = n_tokens // _NW
    n_chunks = n_per_w // chunk

    mesh = plsc.VectorSubcoreMesh(core_axis_name="c", subcore_axis_name="s")

    @functools.partial(
        pl.kernel,
        out_type=jax.ShapeDtypeStruct((n_tokens, hidden), jnp.float32),
        mesh=mesh,
        scratch_types=[
            pltpu.VMEM((chunk,), jnp.int32),
            pltpu.VMEM((chunk, hidden), jnp.float32),
            pltpu.SemaphoreType.DMA,
        ],
    )
    def sc_gather(idx_hbm, table_hbm, x_hbm, idx_v, rows_v, sem):
        wid = lax.axis_index("s") * _NC + lax.axis_index("c")
        base = wid * n_per_w
        for c in range(n_chunks):
            off = base + c * chunk
            pltpu.sync_copy(idx_hbm.at[pl.ds(off, chunk)], idx_v)
            pltpu.async_copy(table_hbm.at[idx_v], rows_v, sem).wait()
            pltpu.sync_copy(rows_v, x_hbm.at[pl.ds(off, chunk)])

    return sc_gather


def _tc_matmul_body(x_ref, w_ref, b_ref, o_ref):
    acc = lax.dot_general(
        w_ref[...],
        x_ref[0],
        (((0,), (1,)), ((), ())),
        preferred_element_type=jnp.float32,
    )
    o_ref[0] = acc + b_ref[...]


def _tc_matmul_t(x_t, w, b_col):
    seqlen, bsz, hidden = x_t.shape
    vocab = w.shape[1]
    return pl.pallas_call(
        _tc_matmul_body,
        grid=(seqlen,),
        in_specs=[
            pl.BlockSpec((1, bsz, hidden), lambda i: (i, 0, 0)),
            pl.BlockSpec((hidden, vocab), lambda i: (0, 0)),
            pl.BlockSpec((vocab, 1), lambda i: (0, 0)),
        ],
        out_specs=pl.BlockSpec((1, vocab, bsz), lambda i: (i, 0, 0)),
        out_shape=jax.ShapeDtypeStruct((seqlen, vocab, bsz), jnp.float32),
    )(x_t, w, b_col)


@jax.jit
def kernel(input_ids, embedding, W, b):
    bsz, seqlen = input_ids.shape
    vocab, hidden = embedding.shape
    n_tokens = bsz * seqlen

    ids_t = input_ids.T.reshape(-1).astype(jnp.int32)
    x_t = _sc_gather_fn(n_tokens, hidden, chunk=80)(ids_t, embedding)
    out_t = _tc_matmul_t(x_t.reshape(seqlen, bsz, hidden), W, b.reshape(-1, 1))
    return jnp.transpose(out_t, (2, 0, 1))


# speedup vs baseline: 3.5599x; 1.0950x over previous
import functools

import jax
import jax.numpy as jnp
from jax import lax
from jax.experimental import pallas as pl
from jax.experimental.pallas import tpu as pltpu
from jax.experimental.pallas import tpu_sc as plsc

_NC = 2
_NS = 16
_NW = _NC * _NS


def _sc_gather_fn(n_tokens, hidden):
    n_per_w = n_tokens // _NW
    sizes = []
    rem = n_per_w
    while rem:
        sz = min(128, rem)
        sizes.append(sz)
        rem -= sz
    offs = [sum(sizes[:i]) for i in range(len(sizes))]

    mesh = plsc.VectorSubcoreMesh(core_axis_name="c", subcore_axis_name="s")

    @functools.partial(
        pl.kernel,
        out_type=jax.ShapeDtypeStruct((n_tokens, hidden), jnp.float32),
        mesh=mesh,
        scratch_types=[
            pltpu.VMEM((n_per_w,), jnp.int32),
            pltpu.VMEM((128, hidden), jnp.float32),
            pltpu.VMEM((128, hidden), jnp.float32),
            pltpu.SemaphoreType.DMA,
            pltpu.SemaphoreType.DMA,
        ],
    )
    def sc_gather(idx_hbm, table_hbm, x_hbm, idx_v, rows0, rows1, sem0, sem1):
        wid = lax.axis_index("s") * _NC + lax.axis_index("c")
        base = wid * n_per_w
        pltpu.sync_copy(idx_hbm.at[pl.ds(base, n_per_w)], idx_v)
        rows = (rows0, rows1)
        sems = (sem0, sem1)
        n = len(sizes)

        def start(c):
            sz = sizes[c]
            return pltpu.async_copy(
                table_hbm.at[idx_v.at[pl.ds(offs[c], sz)]],
                rows[c % 2].at[pl.ds(0, sz)],
                sems[c % 2],
            )

        pending = start(0)
        for c in range(n):
            nxt = start(c + 1) if c + 1 < n else None
            pending.wait()
            pltpu.sync_copy(
                rows[c % 2].at[pl.ds(0, sizes[c])],
                x_hbm.at[pl.ds(base + offs[c], sizes[c])],
            )
            pending = nxt

    return sc_gather


def _tc_matmul_body(x_ref, w_ref, b_ref, o_ref):
    acc = lax.dot_general(
        w_ref[...],
        x_ref[0],
        (((0,), (1,)), ((), ())),
        preferred_element_type=jnp.float32,
    )
    o_ref[0] = acc + b_ref[...]


def _tc_matmul_t(x_t, w, b_col):
    seqlen, bsz, hidden = x_t.shape
    vocab = w.shape[1]
    return pl.pallas_call(
        _tc_matmul_body,
        grid=(seqlen,),
        in_specs=[
            pl.BlockSpec((1, bsz, hidden), lambda i: (i, 0, 0)),
            pl.BlockSpec((hidden, vocab), lambda i: (0, 0)),
            pl.BlockSpec((vocab, 1), lambda i: (0, 0)),
        ],
        out_specs=pl.BlockSpec((1, vocab, bsz), lambda i: (i, 0, 0)),
        out_shape=jax.ShapeDtypeStruct((seqlen, vocab, bsz), jnp.float32),
    )(x_t, w, b_col)


@jax.jit
def kernel(input_ids, embedding, W, b):
    bsz, seqlen = input_ids.shape
    vocab, hidden = embedding.shape
    n_tokens = bsz * seqlen

    ids_t = input_ids.T.reshape(-1).astype(jnp.int32)
    x_t = _sc_gather_fn(n_tokens, hidden)(ids_t, embedding)
    out_t = _tc_matmul_t(x_t.reshape(seqlen, bsz, hidden), W, b.reshape(-1, 1))
    return jnp.transpose(out_t, (2, 0, 1))


# speedup vs baseline: 3.5825x; 1.0063x over previous
import functools

import jax
import jax.numpy as jnp
from jax import lax
from jax.experimental import pallas as pl
from jax.experimental.pallas import tpu as pltpu
from jax.experimental.pallas import tpu_sc as plsc

_NC = 2
_NS = 16
_NW = _NC * _NS


def _sc_gather_fn(n_tokens, hidden):
    n_per_w = n_tokens // _NW
    sizes = []
    rem = n_per_w
    while rem:
        sz = min(128, rem)
        sizes.append(sz)
        rem -= sz
    offs = [sum(sizes[:i]) for i in range(len(sizes))]

    mesh = plsc.VectorSubcoreMesh(core_axis_name="c", subcore_axis_name="s")

    @functools.partial(
        pl.kernel,
        out_type=jax.ShapeDtypeStruct((n_tokens, hidden), jnp.float32),
        mesh=mesh,
        scratch_types=[
            pltpu.VMEM((n_per_w,), jnp.int32),
            pltpu.VMEM((128, hidden), jnp.float32),
            pltpu.VMEM((128, hidden), jnp.float32),
            pltpu.SemaphoreType.DMA,
            pltpu.SemaphoreType.DMA,
        ],
    )
    def sc_gather(idx_hbm, table_hbm, x_hbm, idx_v, rows0, rows1, sem0, sem1):
        wid = lax.axis_index("s") * _NC + lax.axis_index("c")
        base = wid * n_per_w
        pltpu.sync_copy(idx_hbm.at[pl.ds(base, n_per_w)], idx_v)
        rows = (rows0, rows1)
        sems = (sem0, sem1)
        n = len(sizes)

        def start(c):
            sz = sizes[c]
            return pltpu.async_copy(
                table_hbm.at[idx_v.at[pl.ds(offs[c], sz)]],
                rows[c % 2].at[pl.ds(0, sz)],
                sems[c % 2],
            )

        pending = start(0)
        for c in range(n):
            nxt = start(c + 1) if c + 1 < n else None
            pending.wait()
            pltpu.sync_copy(
                rows[c % 2].at[pl.ds(0, sizes[c])],
                x_hbm.at[pl.ds(base + offs[c], sizes[c])],
            )
            pending = nxt

    return sc_gather


def _tc_matmul_body(x_ref, w_ref, b_ref, o_ref):
    acc = lax.dot_general(
        w_ref[...],
        x_ref[0],
        (((0,), (1,)), ((), ())),
        preferred_element_type=jnp.float32,
    )
    o_ref[0] = acc + b_ref[...]


def _tc_matmul_body_aliased(x_ref, w_ref, b_ref, prev_ref, o_ref):
    del prev_ref
    _tc_matmul_body(x_ref, w_ref, b_ref, o_ref)


def _tc_matmul_t_slice(x_t, w, b_col, prev, l_off, seqlen_total):
    n_l, bsz, hidden = x_t.shape
    vocab = w.shape[1]
    specs = [
        pl.BlockSpec((1, bsz, hidden), lambda i: (i, 0, 0)),
        pl.BlockSpec((hidden, vocab), lambda i: (0, 0)),
        pl.BlockSpec((vocab, 1), lambda i: (0, 0)),
    ]
    args = [x_t, w, b_col]
    body = _tc_matmul_body
    aliases = {}
    if prev is not None:
        specs.append(pl.BlockSpec(memory_space=pl.ANY))
        args.append(prev)
        body = _tc_matmul_body_aliased
        aliases = {3: 0}
    return pl.pallas_call(
        body,
        grid=(n_l,),
        in_specs=specs,
        out_specs=pl.BlockSpec(
            (1, vocab, bsz), lambda i, l_off=l_off: (i + l_off, 0, 0)
        ),
        out_shape=jax.ShapeDtypeStruct((seqlen_total, vocab, bsz), jnp.float32),
        input_output_aliases=aliases,
    )(*args)


@jax.jit
def kernel(input_ids, embedding, W, b):
    bsz, seqlen = input_ids.shape
    vocab, hidden = embedding.shape
    n_tokens = bsz * seqlen

    ids_t = input_ids.T.reshape(-1).astype(jnp.int32)
    n_split = 2
    l_part = seqlen // n_split
    tok_part = l_part * bsz
    gather = _sc_gather_fn(tok_part, hidden)
    b_col = b.reshape(-1, 1)
    xs = [
        gather(lax.dynamic_slice_in_dim(ids_t, s * tok_part, tok_part), embedding)
        for s in range(n_split)
    ]
    out_t = None
    for s in range(n_split):
        out_t = _tc_matmul_t_slice(
            xs[s].reshape(l_part, bsz, hidden), W, b_col, out_t,
            s * l_part, seqlen,
        )
    return jnp.transpose(out_t, (2, 0, 1))


# speedup vs baseline: 3.5850x; 1.0007x over previous
import functools

import jax
import jax.numpy as jnp
from jax import lax
from jax.experimental import pallas as pl
from jax.experimental.pallas import tpu as pltpu
from jax.experimental.pallas import tpu_sc as plsc

_NC = 2
_NS = 16
_NW = _NC * _NS


def _sc_gather_fn(n_tokens, hidden):
    n_per_w = n_tokens // _NW
    sizes = []
    rem = n_per_w
    while rem:
        sz = min(128, rem)
        sizes.append(sz)
        rem -= sz
    offs = [sum(sizes[:i]) for i in range(len(sizes))]

    mesh = plsc.VectorSubcoreMesh(core_axis_name="c", subcore_axis_name="s")

    @functools.partial(
        pl.kernel,
        out_type=jax.ShapeDtypeStruct((n_tokens, hidden), jnp.float32),
        mesh=mesh,
        scratch_types=[
            pltpu.VMEM((n_per_w,), jnp.int32),
            pltpu.VMEM((128, hidden), jnp.float32),
            pltpu.VMEM((128, hidden), jnp.float32),
            pltpu.SemaphoreType.DMA,
            pltpu.SemaphoreType.DMA,
        ],
    )
    def sc_gather(idx_hbm, table_hbm, x_hbm, idx_v, rows0, rows1, sem0, sem1):
        wid = lax.axis_index("s") * _NC + lax.axis_index("c")
        base = wid * n_per_w
        pltpu.sync_copy(idx_hbm.at[pl.ds(base, n_per_w)], idx_v)
        rows = (rows0, rows1)
        sems = (sem0, sem1)
        n = len(sizes)

        def start(c):
            sz = sizes[c]
            return pltpu.async_copy(
                table_hbm.at[idx_v.at[pl.ds(offs[c], sz)]],
                rows[c % 2].at[pl.ds(0, sz)],
                sems[c % 2],
            )

        pending = start(0)
        for c in range(n):
            nxt = start(c + 1) if c + 1 < n else None
            pending.wait()
            pltpu.sync_copy(
                rows[c % 2].at[pl.ds(0, sizes[c])],
                x_hbm.at[pl.ds(base + offs[c], sizes[c])],
            )
            pending = nxt

    return sc_gather


def _tc_matmul_body(x_ref, w_ref, b_ref, o_ref):
    acc = lax.dot_general(
        w_ref[...].astype(jnp.bfloat16),
        x_ref[0].astype(jnp.bfloat16),
        (((0,), (1,)), ((), ())),
        preferred_element_type=jnp.float32,
    )
    o_ref[0] = acc + b_ref[...]


def _tc_matmul_body_aliased(x_ref, w_ref, b_ref, prev_ref, o_ref):
    del prev_ref
    _tc_matmul_body(x_ref, w_ref, b_ref, o_ref)


def _tc_matmul_t_slice(x_t, w, b_col, prev, l_off, seqlen_total):
    n_l, bsz, hidden = x_t.shape
    vocab = w.shape[1]
    specs = [
        pl.BlockSpec((1, bsz, hidden), lambda i: (i, 0, 0)),
        pl.BlockSpec((hidden, vocab), lambda i: (0, 0)),
        pl.BlockSpec((vocab, 1), lambda i: (0, 0)),
    ]
    args = [x_t, w, b_col]
    body = _tc_matmul_body
    aliases = {}
    if prev is not None:
        specs.append(pl.BlockSpec(memory_space=pl.ANY))
        args.append(prev)
        body = _tc_matmul_body_aliased
        aliases = {3: 0}
    return pl.pallas_call(
        body,
        grid=(n_l,),
        in_specs=specs,
        out_specs=pl.BlockSpec(
            (1, vocab, bsz), lambda i, l_off=l_off: (i + l_off, 0, 0)
        ),
        out_shape=jax.ShapeDtypeStruct((seqlen_total, vocab, bsz), jnp.float32),
        input_output_aliases=aliases,
    )(*args)


@jax.jit
def kernel(input_ids, embedding, W, b):
    bsz, seqlen = input_ids.shape
    vocab, hidden = embedding.shape
    n_tokens = bsz * seqlen

    ids_t = input_ids.T.reshape(-1).astype(jnp.int32)
    n_split = 2
    l_part = seqlen // n_split
    tok_part = l_part * bsz
    gather = _sc_gather_fn(tok_part, hidden)
    b_col = b.reshape(-1, 1)
    xs = [
        gather(lax.dynamic_slice_in_dim(ids_t, s * tok_part, tok_part), embedding)
        for s in range(n_split)
    ]
    out_t = None
    for s in range(n_split):
        out_t = _tc_matmul_t_slice(
            xs[s].reshape(l_part, bsz, hidden), W, b_col, out_t,
            s * l_part, seqlen,
        )
    return jnp.transpose(out_t, (2, 0, 1))
